# baseline (speedup 1.0000x reference)
import functools

import jax
import jax.numpy as jnp
from jax.experimental import pallas as pl
from jax.experimental.pallas import tpu as pltpu

_TEMPERATURE = 0.07
_BASE_TEMPERATURE = 0.07


def _row_kernel(xi_ref, xf_ref, labr_ref, labc_ref, out_ref, *, n, bi):
    i = pl.program_id(0)
    xi = xi_ref[...]
    xf = xf_ref[...]
    labr = labr_ref[...]
    labc = labc_ref[...]

    gram = jax.lax.dot_general(
        xi, xf, (((1,), (1,)), ((), ())),
        preferred_element_type=jnp.float32,
        precision=jax.lax.Precision.HIGHEST)
    sq_i = jnp.sum(xi * xi, axis=1, keepdims=True)
    ones_row = jnp.ones((1, xf.shape[1]), jnp.float32)
    sq_f = jax.lax.dot_general(
        ones_row, xf * xf, (((1,), (1,)), ((), ())),
        preferred_element_type=jnp.float32,
        precision=jax.lax.Precision.HIGHEST)
    d2 = jnp.maximum(sq_i + sq_f - 2.0 * gram, 0.0)
    dist = jnp.sqrt(d2)

    cols = jax.lax.broadcasted_iota(jnp.int32, (bi, n), 1)
    rows = jax.lax.broadcasted_iota(jnp.int32, (bi, n), 0) + i * bi
    offdiag = cols != rows

    logits = dist * (-1.0 / _TEMPERATURE)
    w = jnp.where(offdiag, jnp.exp(logits), 0.0)
    th = jnp.abs(labc - labr)

    logit_rowsum = jnp.sum(jnp.where(offdiag, logits, 0.0),
                           axis=1, keepdims=True)

    outer = min(128, n)
    n_free = n // outer

    def body(r0, acc):
        labr_r = pltpu.roll(labr, n - r0, axis=1)
        d_r = jnp.abs(labc - labr_r)
        w_r = pltpu.roll(w, n - r0, axis=1)
        cols_out = []
        for c in range(n_free):
            lo = c * outer
            th_c = th[:, lo:lo + outer]
            s = acc[:, lo:lo + outer]
            for k in range(n_free):
                src = ((c + k) % n_free) * outer
                dk = d_r[:, src:src + outer]
                wk = w_r[:, src:src + outer]
                s = s + jnp.where(dk >= th_c, wk, 0.0)
            cols_out.append(s)
        return jnp.concatenate(cols_out, axis=1)

    acc = jax.lax.fori_loop(0, outer, body, jnp.zeros((bi, n), jnp.float32))

    neg = jnp.where(offdiag, acc, 1.0)
    logneg_rowsum = jnp.sum(jnp.log(neg), axis=1, keepdims=True)
    out_ref[...] = logit_rowsum - logneg_rowsum


def kernel(features, labels):
    b, v, d = features.shape
    n = b * v
    cf = features.transpose(1, 0, 2).reshape(n, d)
    lab = jnp.tile(labels, v)
    lab_row = lab.reshape(1, n)
    lab_col = lab.reshape(n, 1)

    bi = 256 if n % 256 == 0 else (128 if n % 128 == 0 else n)
    grid = (n // bi,)

    rows = pl.pallas_call(
        functools.partial(_row_kernel, n=n, bi=bi),
        grid=grid,
        in_specs=[
            pl.BlockSpec((bi, d), lambda i: (i, 0)),
            pl.BlockSpec((n, d), lambda i: (0, 0)),
            pl.BlockSpec((1, n), lambda i: (0, 0)),
            pl.BlockSpec((bi, 1), lambda i: (i, 0)),
        ],
        out_specs=pl.BlockSpec((bi, 1), lambda i: (i, 0)),
        out_shape=jax.ShapeDtypeStruct((n, 1), jnp.float32),
        compiler_params=pltpu.CompilerParams(
            dimension_semantics=("parallel",),
            vmem_limit_bytes=48 * 1024 * 1024,
        ),
    )(cf, cf, lab_row, lab_col)

    mean_log_prob_pos = jnp.sum(rows) / (n * (n - 1))
    return -(_TEMPERATURE / _BASE_TEMPERATURE) * mean_log_prob_pos


# speedup vs baseline: 1.0156x; 1.0156x over previous
import functools

import jax
import jax.numpy as jnp
from jax.experimental import pallas as pl
from jax.experimental.pallas import tpu as pltpu

_TEMPERATURE = 0.07
_BASE_TEMPERATURE = 0.07


def _row_kernel(xi_ref, xf_ref, labr_ref, labc_ref, out_ref, *, n, bi):
    i = pl.program_id(0)
    xi = xi_ref[...]
    xf = xf_ref[...]
    labr = labr_ref[...]
    labc = labc_ref[...]

    gram = jax.lax.dot_general(
        xi, xf, (((1,), (1,)), ((), ())),
        preferred_element_type=jnp.float32,
        precision=jax.lax.Precision.HIGHEST)
    sq_i = jnp.sum(xi * xi, axis=1, keepdims=True)
    ones_row = jnp.ones((1, xf.shape[1]), jnp.float32)
    sq_f = jax.lax.dot_general(
        ones_row, xf * xf, (((1,), (1,)), ((), ())),
        preferred_element_type=jnp.float32,
        precision=jax.lax.Precision.HIGHEST)
    d2 = jnp.maximum(sq_i + sq_f - 2.0 * gram, 0.0)
    dist = jnp.sqrt(d2)

    cols = jax.lax.broadcasted_iota(jnp.int32, (bi, n), 1)
    rows = jax.lax.broadcasted_iota(jnp.int32, (bi, n), 0) + i * bi
    offdiag = cols != rows

    logits = dist * (-1.0 / _TEMPERATURE)
    w = jnp.where(offdiag, jnp.exp(logits), 0.0)
    th = jnp.abs(labc - labr)

    logit_rowsum = jnp.sum(jnp.where(offdiag, logits, 0.0),
                           axis=1, keepdims=True)

    outer = min(128, n)
    n_free = n // outer

    def body(r0, acc):
        labr_r = pltpu.roll(labr, n - r0, axis=1)
        d_r = jnp.abs(labc - labr_r)
        w_r = pltpu.roll(w, n - r0, axis=1)
        cols_out = []
        for c in range(n_free):
            lo = c * outer
            th_c = th[:, lo:lo + outer]
            s = acc[:, lo:lo + outer]
            for k in range(n_free):
                src = ((c + k) % n_free) * outer
                dk = d_r[:, src:src + outer]
                wk = w_r[:, src:src + outer]
                s = s + jnp.where(dk >= th_c, wk, 0.0)
            cols_out.append(s)
        return jnp.concatenate(cols_out, axis=1)

    acc = jax.lax.fori_loop(0, outer, body, jnp.zeros((bi, n), jnp.float32))

    neg = jnp.where(offdiag, acc, 1.0)
    logneg_rowsum = jnp.sum(jnp.log(neg), axis=1, keepdims=True)
    out_ref[...] = logit_rowsum - logneg_rowsum


def kernel(features, labels):
    b, v, d = features.shape
    n = b * v
    cf = features.transpose(1, 0, 2).reshape(n, d)
    lab = jnp.tile(labels, v)
    lab_row = lab.reshape(1, n)
    lab_col = lab.reshape(n, 1)

    bi = 128 if n % 128 == 0 else n
    grid = (n // bi,)

    rows = pl.pallas_call(
        functools.partial(_row_kernel, n=n, bi=bi),
        grid=grid,
        in_specs=[
            pl.BlockSpec((bi, d), lambda i: (i, 0)),
            pl.BlockSpec((n, d), lambda i: (0, 0)),
            pl.BlockSpec((1, n), lambda i: (0, 0)),
            pl.BlockSpec((bi, 1), lambda i: (i, 0)),
        ],
        out_specs=pl.BlockSpec((bi, 1), lambda i: (i, 0)),
        out_shape=jax.ShapeDtypeStruct((n, 1), jnp.float32),
        compiler_params=pltpu.CompilerParams(
            dimension_semantics=("parallel",),
            vmem_limit_bytes=48 * 1024 * 1024,
        ),
    )(cf, cf, lab_row, lab_col)

    mean_log_prob_pos = jnp.sum(rows) / (n * (n - 1))
    return -(_TEMPERATURE / _BASE_TEMPERATURE) * mean_log_prob_pos
